# Initial kernel scaffold; baseline (speedup 1.0000x reference)
import jax
import jax.numpy as jnp
from jax.experimental import pallas as pl


def kernel(x, edge_index, W0, b0, Wl1, bl1, Wr1, g1, be1, Wl2, bl2, Wr2, g2, be2):
    raise NotImplementedError("write your pallas kernel here")


# speedup vs baseline: 6.2199x; 6.2199x over previous
import functools

import jax
import jax.numpy as jnp
from jax import lax
from jax.experimental import pallas as pl
from jax.experimental.pallas import tpu as pltpu
from jax.experimental.pallas import tpu_sc as plsc

N = 10000
E = 320000
F_IN = 128
H = 64
D = 128

C = 128
K = 80
NW = 32
EW = C * K
EPAD = EW * NW
NP = 10240
RPT = NP // 16
NBUF = 4

_f32 = jnp.float32


def _make_sc_agg(with_counts: bool):
  mesh = plsc.VectorSubcoreMesh(core_axis_name="c", subcore_axis_name="s")

  out_type = [jax.ShapeDtypeStruct((2, NP, H), _f32)]
  if with_counts:
    out_type.append(jax.ShapeDtypeStruct((2, NP, 16), _f32))

  scratch = [
      pltpu.VMEM((K, C), jnp.int32),
      pltpu.VMEM((K, C), jnp.int32),
      pltpu.VMEM((NBUF, C, H), _f32),
      pltpu.VMEM((C, H), _f32),
      pltpu.VMEM_SHARED((NP, H), _f32),
      pltpu.SemaphoreType.DMA((NBUF,)),
  ]
  if with_counts:
    scratch += [
        pltpu.VMEM((C, 16), _f32),
        pltpu.VMEM((C, 16), _f32),
        pltpu.VMEM_SHARED((NP, 16), _f32),
    ]

  def body(*refs):
    if with_counts:
      (table, srci, dsti, z64, o16, z16,
       sum_out, cnt_out,
       src_v, dst_v, rows, zbuf, acc, sems, obuf, zcbuf, cacc) = refs
    else:
      (table, srci, dsti, z64,
       sum_out,
       src_v, dst_v, rows, zbuf, acc, sems) = refs

    c = lax.axis_index("c")
    s = lax.axis_index("s")
    w = c * 16 + s

    pltpu.sync_copy(srci.at[w], src_v)
    pltpu.sync_copy(dsti.at[w], dst_v)
    pltpu.sync_copy(z64, zbuf)
    if with_counts:
      pltpu.sync_copy(o16, obuf)
      pltpu.sync_copy(z16, zcbuf)

    for j in range(RPT // C):
      off = s * RPT + j * C
      pltpu.sync_copy(zbuf, acc.at[pl.ds(off, C)])
      if with_counts:
        pltpu.sync_copy(zcbuf, cacc.at[pl.ds(off, C)])
    plsc.subcore_barrier()

    for b in range(NBUF):
      pltpu.async_copy(table.at[src_v.at[b]], rows.at[b], sems.at[b])

    @pl.loop(0, K, step=NBUF)
    def _(g):
      for b in range(NBUF):
        j = g + b
        pltpu.make_async_copy(table.at[src_v.at[j]], rows.at[b],
                              sems.at[b]).wait()
        pltpu.sync_copy(rows.at[b], acc.at[dst_v.at[j]], add=True)
        if with_counts:
          pltpu.sync_copy(obuf, cacc.at[dst_v.at[j]], add=True)
        nxt = j + NBUF

        @pl.when(nxt < K)
        def _():
          pltpu.async_copy(table.at[src_v.at[nxt]], rows.at[b], sems.at[b])

    plsc.subcore_barrier()

    pltpu.sync_copy(acc.at[pl.ds(s * RPT, RPT)],
                    sum_out.at[c, pl.ds(s * RPT, RPT)])
    if with_counts:
      pltpu.sync_copy(cacc.at[pl.ds(s * RPT, RPT)],
                      cnt_out.at[c, pl.ds(s * RPT, RPT)])

  return pl.kernel(body, out_type=out_type, mesh=mesh,
                   scratch_types=scratch,
                   compiler_params=pltpu.CompilerParams(
                       use_tc_tiling_on_sc=False),
                   name="sc_agg_cnt" if with_counts else "sc_agg")


_sc_agg_cnt = _make_sc_agg(True)
_sc_agg = _make_sc_agg(False)

_BN = 1000


def _tc_proj_body(x_ref, w_ref, b_ref, o_ref):
  z = jnp.dot(x_ref[...], w_ref[...], preferred_element_type=_f32)
  o_ref[...] = jnp.maximum(z + b_ref[...], 0.0)


def _tc_combine_body(p_ref, c_ref, h_ref, wl_ref, wr_ref, b_ref, g_ref,
                     be_ref, o_ref):
  summed = p_ref[0] + p_ref[1]
  cnt = c_ref[0, :, 0:1] + c_ref[1, :, 0:1]
  mean = summed / jnp.maximum(cnt, 1.0)
  z = (jnp.dot(mean, wl_ref[...], preferred_element_type=_f32)
       + jnp.dot(h_ref[...], wr_ref[...], preferred_element_type=_f32)
       + b_ref[...])
  mu = jnp.mean(z, axis=-1, keepdims=True)
  var = jnp.mean((z - mu) ** 2, axis=-1, keepdims=True)
  zn = (z - mu) * lax.rsqrt(var + 1e-5) * g_ref[...] + be_ref[...]
  o_ref[...] = jnp.maximum(zn, 0.0)


def _tc_proj(x, w_t, b):
  return pl.pallas_call(
      _tc_proj_body,
      grid=(N // _BN,),
      in_specs=[
          pl.BlockSpec((_BN, F_IN), lambda i: (i, 0)),
          pl.BlockSpec((F_IN, H), lambda i: (0, 0)),
          pl.BlockSpec((1, H), lambda i: (0, 0)),
      ],
      out_specs=pl.BlockSpec((_BN, H), lambda i: (i, 0)),
      out_shape=jax.ShapeDtypeStruct((N, H), _f32),
  )(x, w_t, b)


def _tc_combine(p, cnt, h, wl_t, wr_t, b, g, be, d_out):
  h_in = h.shape[-1]
  return pl.pallas_call(
      _tc_combine_body,
      grid=(N // _BN,),
      in_specs=[
          pl.BlockSpec((2, _BN, h_in), lambda i: (0, i, 0)),
          pl.BlockSpec((2, _BN, 16), lambda i: (0, i, 0)),
          pl.BlockSpec((_BN, h_in), lambda i: (i, 0)),
          pl.BlockSpec((h_in, d_out), lambda i: (0, 0)),
          pl.BlockSpec((h_in, d_out), lambda i: (0, 0)),
          pl.BlockSpec((1, d_out), lambda i: (0, 0)),
          pl.BlockSpec((1, d_out), lambda i: (0, 0)),
          pl.BlockSpec((1, d_out), lambda i: (0, 0)),
      ],
      out_specs=pl.BlockSpec((_BN, d_out), lambda i: (i, 0)),
      out_shape=jax.ShapeDtypeStruct((N, d_out), _f32),
  )(p, cnt, h, wl_t, wr_t, b, g, be)


@jax.jit
def kernel(x, edge_index, W0, b0, Wl1, bl1, Wr1, g1, be1, Wl2, bl2, Wr2,
           g2, be2):
  src = edge_index[0]
  dst = edge_index[1]
  pad = EPAD - E
  srcp = jnp.concatenate([src, jnp.zeros((pad,), jnp.int32)]).reshape(
      NW, K, C)
  dstp = jnp.concatenate([dst, jnp.full((pad,), N, jnp.int32)]).reshape(
      NW, K, C)
  z64 = jnp.zeros((C, H), _f32)
  o16 = jnp.ones((C, 16), _f32)
  z16 = jnp.zeros((C, 16), _f32)

  h0 = _tc_proj(x, W0.T, b0.reshape(1, H))
  sums1, cnts = _sc_agg_cnt(h0, srcp, dstp, z64, o16, z16)
  h1 = _tc_combine(sums1, cnts, h0, Wl1.T, Wr1.T, bl1.reshape(1, H),
                   g1.reshape(1, H), be1.reshape(1, H), H)
  (sums2,) = _sc_agg(h1, srcp, dstp, z64)
  out = _tc_combine(sums2, cnts, h1, Wl2.T, Wr2.T, bl2.reshape(1, D),
                    g2.reshape(1, D), be2.reshape(1, D), D)
  return out


# speedup vs baseline: 6.3479x; 1.0206x over previous
import functools

import jax
import jax.numpy as jnp
from jax import lax
from jax.experimental import pallas as pl
from jax.experimental.pallas import tpu as pltpu
from jax.experimental.pallas import tpu_sc as plsc

N = 10000
E = 320000
F_IN = 128
H = 64
D = 128

C = 128
K = 80
NW = 32
EW = C * K
EPAD = EW * NW
NP = 10240
RPT = NP // 16
NBUF = 4

_f32 = jnp.float32


def _make_sc_agg(with_counts: bool):
  mesh = plsc.VectorSubcoreMesh(core_axis_name="c", subcore_axis_name="s")

  out_type = [jax.ShapeDtypeStruct((2, NP, H), _f32)]
  if with_counts:
    out_type.append(jax.ShapeDtypeStruct((2, NP, 16), _f32))

  scratch = [
      pltpu.VMEM((K, C), jnp.int32),
      pltpu.VMEM((K, C), jnp.int32),
      pltpu.VMEM((NBUF, C, H), _f32),
      pltpu.VMEM((C, H), _f32),
      pltpu.VMEM_SHARED((NP, H), _f32),
      pltpu.SemaphoreType.DMA((NBUF,)),
      pltpu.SemaphoreType.DMA((NBUF,)),
  ]
  if with_counts:
    scratch += [
        pltpu.VMEM((C, 16), _f32),
        pltpu.VMEM((C, 16), _f32),
        pltpu.VMEM_SHARED((NP, 16), _f32),
        pltpu.SemaphoreType.DMA,
    ]

  def body(*refs):
    if with_counts:
      (table, srci, dsti, z64, o16, z16,
       sum_out, cnt_out,
       src_v, dst_v, rows, zbuf, acc, sems, ssems,
       obuf, zcbuf, cacc, csem) = refs
    else:
      (table, srci, dsti, z64,
       sum_out,
       src_v, dst_v, rows, zbuf, acc, sems, ssems) = refs

    c = lax.axis_index("c")
    s = lax.axis_index("s")
    w = c * 16 + s

    pltpu.sync_copy(srci.at[w], src_v)
    pltpu.sync_copy(dsti.at[w], dst_v)
    pltpu.sync_copy(z64, zbuf)
    if with_counts:
      pltpu.sync_copy(o16, obuf)
      pltpu.sync_copy(z16, zcbuf)

    for j in range(RPT // C):
      off = s * RPT + j * C
      pltpu.sync_copy(zbuf, acc.at[pl.ds(off, C)])
      if with_counts:
        pltpu.sync_copy(zcbuf, cacc.at[pl.ds(off, C)])
    plsc.subcore_barrier()

    for b in range(NBUF):
      pltpu.async_copy(table.at[src_v.at[b]], rows.at[b], sems.at[b])

    @pl.loop(0, K, step=NBUF)
    def _(g):
      for b in range(NBUF):
        j = g + b
        pltpu.make_async_copy(table.at[src_v.at[j]], rows.at[b],
                              sems.at[b]).wait()
        pltpu.async_copy(rows.at[b], acc.at[dst_v.at[j]], ssems.at[b],
                         add=True)
        if with_counts:
          pltpu.async_copy(obuf, cacc.at[dst_v.at[j]], csem, add=True)

          @pl.when(j >= 2)
          def _():
            pltpu.make_async_copy(obuf, cacc.at[dst_v.at[j]], csem).wait()

        nxt = j + NBUF

        @pl.when(nxt < K)
        def _():
          pltpu.make_async_copy(rows.at[b], acc.at[dst_v.at[j]],
                                ssems.at[b]).wait()
          pltpu.async_copy(table.at[src_v.at[nxt]], rows.at[b], sems.at[b])

    for b in range(NBUF):
      pltpu.make_async_copy(rows.at[b], acc.at[dst_v.at[b]],
                            ssems.at[b]).wait()
    if with_counts:
      for _ in range(2):
        pltpu.make_async_copy(obuf, cacc.at[dst_v.at[0]], csem).wait()

    plsc.subcore_barrier()

    pltpu.sync_copy(acc.at[pl.ds(s * RPT, RPT)],
                    sum_out.at[c, pl.ds(s * RPT, RPT)])
    if with_counts:
      pltpu.sync_copy(cacc.at[pl.ds(s * RPT, RPT)],
                      cnt_out.at[c, pl.ds(s * RPT, RPT)])

  return pl.kernel(body, out_type=out_type, mesh=mesh,
                   scratch_types=scratch,
                   compiler_params=pltpu.CompilerParams(
                       use_tc_tiling_on_sc=False),
                   name="sc_agg_cnt" if with_counts else "sc_agg")


_sc_agg_cnt = _make_sc_agg(True)
_sc_agg = _make_sc_agg(False)

_BN = 1000


def _tc_proj_body(x_ref, w_ref, b_ref, o_ref):
  z = jnp.dot(x_ref[...], w_ref[...], preferred_element_type=_f32)
  o_ref[...] = jnp.maximum(z + b_ref[...], 0.0)


def _tc_combine_body(p_ref, c_ref, h_ref, wl_ref, wr_ref, b_ref, g_ref,
                     be_ref, o_ref):
  summed = p_ref[0] + p_ref[1]
  cnt = c_ref[0, :, 0:1] + c_ref[1, :, 0:1]
  mean = summed / jnp.maximum(cnt, 1.0)
  z = (jnp.dot(mean, wl_ref[...], preferred_element_type=_f32)
       + jnp.dot(h_ref[...], wr_ref[...], preferred_element_type=_f32)
       + b_ref[...])
  mu = jnp.mean(z, axis=-1, keepdims=True)
  var = jnp.mean((z - mu) ** 2, axis=-1, keepdims=True)
  zn = (z - mu) * lax.rsqrt(var + 1e-5) * g_ref[...] + be_ref[...]
  o_ref[...] = jnp.maximum(zn, 0.0)


def _tc_proj(x, w_t, b):
  return pl.pallas_call(
      _tc_proj_body,
      grid=(N // _BN,),
      in_specs=[
          pl.BlockSpec((_BN, F_IN), lambda i: (i, 0)),
          pl.BlockSpec((F_IN, H), lambda i: (0, 0)),
          pl.BlockSpec((1, H), lambda i: (0, 0)),
      ],
      out_specs=pl.BlockSpec((_BN, H), lambda i: (i, 0)),
      out_shape=jax.ShapeDtypeStruct((N, H), _f32),
  )(x, w_t, b)


def _tc_combine(p, cnt, h, wl_t, wr_t, b, g, be, d_out):
  h_in = h.shape[-1]
  return pl.pallas_call(
      _tc_combine_body,
      grid=(N // _BN,),
      in_specs=[
          pl.BlockSpec((2, _BN, h_in), lambda i: (0, i, 0)),
          pl.BlockSpec((2, _BN, 16), lambda i: (0, i, 0)),
          pl.BlockSpec((_BN, h_in), lambda i: (i, 0)),
          pl.BlockSpec((h_in, d_out), lambda i: (0, 0)),
          pl.BlockSpec((h_in, d_out), lambda i: (0, 0)),
          pl.BlockSpec((1, d_out), lambda i: (0, 0)),
          pl.BlockSpec((1, d_out), lambda i: (0, 0)),
          pl.BlockSpec((1, d_out), lambda i: (0, 0)),
      ],
      out_specs=pl.BlockSpec((_BN, d_out), lambda i: (i, 0)),
      out_shape=jax.ShapeDtypeStruct((N, d_out), _f32),
  )(p, cnt, h, wl_t, wr_t, b, g, be)


@jax.jit
def kernel(x, edge_index, W0, b0, Wl1, bl1, Wr1, g1, be1, Wl2, bl2, Wr2,
           g2, be2):
  src = edge_index[0]
  dst = edge_index[1]
  pad = EPAD - E
  srcp = jnp.concatenate([src, jnp.zeros((pad,), jnp.int32)]).reshape(
      NW, K, C)
  pad_dst = N + (jnp.arange(pad, dtype=jnp.int32) % (NP - N))
  dstp = jnp.concatenate([dst, pad_dst]).reshape(NW, K, C)
  z64 = jnp.zeros((C, H), _f32)
  o16 = jnp.ones((C, 16), _f32)
  z16 = jnp.zeros((C, 16), _f32)

  h0 = _tc_proj(x, W0.T, b0.reshape(1, H))
  sums1, cnts = _sc_agg_cnt(h0, srcp, dstp, z64, o16, z16)
  h1 = _tc_combine(sums1, cnts, h0, Wl1.T, Wr1.T, bl1.reshape(1, H),
                   g1.reshape(1, H), be1.reshape(1, H), H)
  (sums2,) = _sc_agg(h1, srcp, dstp, z64)
  out = _tc_combine(sums2, cnts, h1, Wl2.T, Wr2.T, bl2.reshape(1, D),
                    g2.reshape(1, D), be2.reshape(1, D), D)
  return out
